# baseline (speedup 1.0000x reference)
import jax
import jax.numpy as jnp
from jax import lax
from jax.experimental import pallas as pl
from jax.experimental.pallas import tpu as pltpu
from jax.experimental.pallas import tpu_sc as plsc

N_NODES = 10000
N_EDGES = 160000
IN_DIM = 256
OUT_DIM = 256

NC = 2
NS = 16
HALF = IN_DIM // 2
CHUNK = 80
CHUNKS_PER_CORE = 1024
CHUNKS_PER_TILE = 64
E_PAD = NC * CHUNKS_PER_CORE * CHUNK
AGG_ROWS = 10008
TRASH_ROW = N_NODES

IDX_GROUP = 16
GROUPS = [(g, IDX_GROUP) for g in range(0, CHUNKS_PER_TILE, IDX_GROUP)]

SLAB = 632
LAST_SLAB = N_NODES - 15 * SLAB
LAST_BASE = 15 * SLAB

_HI_MASK = -65536


def _unpack_chunk(pk, bb, keep_odd):
    def row(r, carry):
        for w in range(8):
            x = plsc.bitcast(pk[r, pl.ds(w * 16, 16)], jnp.int32)
            lo = plsc.bitcast(jnp.left_shift(x, 16), jnp.float32)
            hi = plsc.bitcast(jnp.bitwise_and(x, _HI_MASK), jnp.float32)
            pk[r, pl.ds(w * 16, 16)] = hi if keep_odd else lo
            bb[r, pl.ds(w * 16, 16)] = lo if keep_odd else hi
        return carry
    lax.fori_loop(0, CHUNK, row, 0)


def _zero_slabs(zrows, agg_sh, sid):
    @pl.when(sid < 15)
    def _():
        pltpu.sync_copy(zrows, agg_sh.at[pl.ds(sid * SLAB, SLAB)])

    @pl.when(sid == 15)
    def _():
        pltpu.sync_copy(zrows.at[pl.ds(0, LAST_SLAB)],
                        agg_sh.at[pl.ds(LAST_BASE, LAST_SLAB)])


def _writeout_slabs(agg_sh, out, cid, sid):
    @pl.when(sid < 15)
    def _():
        pltpu.sync_copy(agg_sh.at[pl.ds(sid * SLAB, SLAB)],
                        out.at[cid, pl.ds(sid * SLAB, SLAB)])

    @pl.when(sid == 15)
    def _():
        pltpu.sync_copy(agg_sh.at[pl.ds(LAST_BASE, LAST_SLAB)],
                        out.at[cid, pl.ds(LAST_BASE, LAST_SLAB)])


def _sc_a_body(pt, srcs, dsts, zrows, agg_out, bounce,
               srcA, srcB, dstA, dstB, pk0, pk1, bb0, bb1, agg_sh,
               g0, g1, s0, s1, w0, w1, iS):
    pks = [pk0, pk1]
    bbs = [bb0, bb1]
    gsems = [g0, g1]
    ssems = [s0, s1]
    wsems = [w0, w1]
    src_pair = [srcA, srcB]
    dst_pair = [dstA, dstB]
    cid = lax.axis_index("c")
    sid = lax.axis_index("s")
    tchunk = cid * CHUNKS_PER_CORE + sid * CHUNKS_PER_TILE
    bchunk = sid * CHUNKS_PER_TILE

    def load_idx(gi):
        gbase, gsize = GROUPS[gi]
        p = gi % 2
        pltpu.async_copy(srcs.at[pl.ds(tchunk + gbase, gsize)],
                         src_pair[p], iS)
        pltpu.async_copy(dsts.at[pl.ds(tchunk + gbase, gsize)],
                         dst_pair[p], iS)

    def wait_idx(gi):
        gbase, gsize = GROUPS[gi]
        p = gi % 2
        pltpu.make_async_copy(srcs.at[pl.ds(tchunk + gbase, gsize)],
                              src_pair[p], iS).wait()
        pltpu.make_async_copy(dsts.at[pl.ds(tchunk + gbase, gsize)],
                              dst_pair[p], iS).wait()

    load_idx(0)
    _zero_slabs(zrows, agg_sh, sid)
    wait_idx(0)
    plsc.subcore_barrier()

    def start_gather(sv, c, b):
        pltpu.async_copy(pt.at[sv.at[c]], pks[b], gsems[b])

    def wait_gather(sv, b):
        pltpu.make_async_copy(pt.at[sv.at[0]], pks[b], gsems[b]).wait()

    def edge_pass(keep_odd):
        for gi, (gbase, gsize) in enumerate(GROUPS):
            sv = src_pair[gi % 2]
            dv = dst_pair[gi % 2]
            if gi + 1 < len(GROUPS):
                load_idx(gi + 1)
            start_gather(sv, 0, 0)
            start_gather(sv, 1, 1)

            def step(jo, carry):
                for b in range(2):
                    c = jo * 2 + b
                    wait_gather(sv, b)
                    _unpack_chunk(pks[b], bbs[b], keep_odd)
                    pltpu.async_copy(pks[b], agg_sh.at[dv.at[c]], ssems[b],
                                     add=True)
                    pltpu.async_copy(
                        bbs[b],
                        bounce.at[cid, pl.ds((bchunk + gbase + c) * CHUNK,
                                             CHUNK)],
                        wsems[b])

                    @pl.when(c + 2 < gsize)
                    def _():
                        pltpu.make_async_copy(pks[b], agg_sh.at[dv.at[0]],
                                              ssems[b]).wait()
                        pltpu.make_async_copy(
                            bbs[b],
                            bounce.at[cid, pl.ds(bchunk * CHUNK, CHUNK)],
                            wsems[b]).wait()
                        start_gather(sv, c + 2, b)
                return carry

            lax.fori_loop(0, gsize // 2, step, 0)
            for b in range(2):
                pltpu.make_async_copy(pks[b], agg_sh.at[dv.at[0]],
                                      ssems[b]).wait()
                pltpu.make_async_copy(
                    bbs[b], bounce.at[cid, pl.ds(bchunk * CHUNK, CHUNK)],
                    wsems[b]).wait()
            if gi + 1 < len(GROUPS):
                wait_idx(gi + 1)

    pl.when(cid == 0)(lambda: edge_pass(False))
    pl.when(cid == 1)(lambda: edge_pass(True))

    plsc.subcore_barrier()
    _writeout_slabs(agg_sh, agg_out, cid, sid)


_sc_a = pl.kernel(
    _sc_a_body,
    out_type=(
        jax.ShapeDtypeStruct((NC, N_NODES, HALF), jnp.float32),
        jax.ShapeDtypeStruct((NC, CHUNKS_PER_CORE * CHUNK, HALF),
                             jnp.float32),
    ),
    mesh=plsc.VectorSubcoreMesh(core_axis_name="c", subcore_axis_name="s",
                                num_cores=NC, num_subcores=NS),
    compiler_params=pltpu.CompilerParams(needs_layout_passes=False),
    scratch_types=[
        pltpu.VMEM((IDX_GROUP, CHUNK), jnp.int32),
        pltpu.VMEM((IDX_GROUP, CHUNK), jnp.int32),
        pltpu.VMEM((IDX_GROUP, CHUNK), jnp.int32),
        pltpu.VMEM((IDX_GROUP, CHUNK), jnp.int32),
        pltpu.VMEM((CHUNK, HALF), jnp.float32),
        pltpu.VMEM((CHUNK, HALF), jnp.float32),
        pltpu.VMEM((CHUNK, HALF), jnp.float32),
        pltpu.VMEM((CHUNK, HALF), jnp.float32),
        pltpu.VMEM_SHARED((AGG_ROWS, HALF), jnp.float32),
    ] + [pltpu.SemaphoreType.DMA] * 7,
)


def _sc_b_body(agg_in, bounce, dsts, agg_out,
               dstA, dstB, rb0, rb1, agg_sh,
               l0, l1, s0, s1, iS):
    rbs = [rb0, rb1]
    lsems = [l0, l1]
    ssems = [s0, s1]
    dst_pair = [dstA, dstB]
    cid = lax.axis_index("c")
    sid = lax.axis_index("s")
    ocore = 1 - cid
    tchunk = ocore * CHUNKS_PER_CORE + sid * CHUNKS_PER_TILE
    bchunk = sid * CHUNKS_PER_TILE

    def load_idx(gi):
        gbase, gsize = GROUPS[gi]
        pltpu.async_copy(dsts.at[pl.ds(tchunk + gbase, gsize)],
                         dst_pair[gi % 2], iS)

    def wait_idx(gi):
        gbase, gsize = GROUPS[gi]
        pltpu.make_async_copy(dsts.at[pl.ds(tchunk + gbase, gsize)],
                              dst_pair[gi % 2], iS).wait()

    load_idx(0)

    @pl.when(sid < 15)
    def _():
        pltpu.sync_copy(agg_in.at[cid, pl.ds(sid * SLAB, SLAB)],
                        agg_sh.at[pl.ds(sid * SLAB, SLAB)])

    @pl.when(sid == 15)
    def _():
        pltpu.sync_copy(agg_in.at[cid, pl.ds(LAST_BASE, LAST_SLAB)],
                        agg_sh.at[pl.ds(LAST_BASE, LAST_SLAB)])

    wait_idx(0)
    plsc.subcore_barrier()

    def start_load(c_global, b):
        pltpu.async_copy(
            bounce.at[ocore, pl.ds((bchunk + c_global) * CHUNK, CHUNK)],
            rbs[b], lsems[b])

    def wait_load(b):
        pltpu.make_async_copy(
            bounce.at[ocore, pl.ds(bchunk * CHUNK, CHUNK)],
            rbs[b], lsems[b]).wait()

    for gi, (gbase, gsize) in enumerate(GROUPS):
        dv = dst_pair[gi % 2]
        if gi + 1 < len(GROUPS):
            load_idx(gi + 1)
        start_load(gbase, 0)
        start_load(gbase + 1, 1)

        def step(jo, carry):
            for b in range(2):
                c = jo * 2 + b
                wait_load(b)
                pltpu.async_copy(rbs[b], agg_sh.at[dv.at[c]], ssems[b],
                                 add=True)

                @pl.when(c + 2 < gsize)
                def _():
                    pltpu.make_async_copy(rbs[b], agg_sh.at[dv.at[0]],
                                          ssems[b]).wait()
                    start_load(gbase + c + 2, b)
            return carry

        lax.fori_loop(0, gsize // 2, step, 0)
        for b in range(2):
            pltpu.make_async_copy(rbs[b], agg_sh.at[dv.at[0]],
                                  ssems[b]).wait()
        if gi + 1 < len(GROUPS):
            wait_idx(gi + 1)

    plsc.subcore_barrier()
    _writeout_slabs(agg_sh, agg_out, cid, sid)


_sc_b = pl.kernel(
    _sc_b_body,
    out_type=jax.ShapeDtypeStruct((NC, N_NODES, HALF), jnp.float32),
    mesh=plsc.VectorSubcoreMesh(core_axis_name="c", subcore_axis_name="s",
                                num_cores=NC, num_subcores=NS),
    scratch_types=[
        pltpu.VMEM((IDX_GROUP, CHUNK), jnp.int32),
        pltpu.VMEM((IDX_GROUP, CHUNK), jnp.int32),
        pltpu.VMEM((CHUNK, HALF), jnp.float32),
        pltpu.VMEM((CHUNK, HALF), jnp.float32),
        pltpu.VMEM_SHARED((AGG_ROWS, HALF), jnp.float32),
    ] + [pltpu.SemaphoreType.DMA] * 5,
)


ROW_BLK = 2000
GRID = N_NODES // ROW_BLK


def _tc_body(f_ref, a0_ref, a1_ref, w1_ref, w2a_ref, w2b_ref, w3_ref,
             out_ref, acc_ref):
    i = pl.program_id(0)
    z = jnp.tanh(
        jnp.dot(f_ref[...], w1_ref[...], preferred_element_type=jnp.float32)
        + jnp.dot(a0_ref[...], w2a_ref[...],
                  preferred_element_type=jnp.float32)
        + jnp.dot(a1_ref[...], w2b_ref[...],
                  preferred_element_type=jnp.float32))
    p = jnp.sum(z, axis=0, keepdims=True)

    @pl.when(i == 0)
    def _():
        acc_ref[...] = p

    @pl.when(i != 0)
    def _():
        acc_ref[...] = acc_ref[...] + p

    @pl.when(i == GRID - 1)
    def _():
        out_ref[...] = jnp.dot(acc_ref[...], w3_ref[...],
                               preferred_element_type=jnp.float32)


_tc_pool = pl.pallas_call(
    _tc_body,
    grid=(GRID,),
    in_specs=[
        pl.BlockSpec((ROW_BLK, IN_DIM), lambda i: (i, 0)),
        pl.BlockSpec((ROW_BLK, HALF), lambda i: (i, 0)),
        pl.BlockSpec((ROW_BLK, HALF), lambda i: (i, 0)),
        pl.BlockSpec((IN_DIM, OUT_DIM), lambda i: (0, 0)),
        pl.BlockSpec((HALF, OUT_DIM), lambda i: (0, 0)),
        pl.BlockSpec((HALF, OUT_DIM), lambda i: (0, 0)),
        pl.BlockSpec((OUT_DIM, OUT_DIM), lambda i: (0, 0)),
    ],
    out_specs=pl.BlockSpec((1, OUT_DIM), lambda i: (0, 0)),
    out_shape=jax.ShapeDtypeStruct((1, OUT_DIM), jnp.float32),
    scratch_shapes=[pltpu.VMEM((1, OUT_DIM), jnp.float32)],
)


@jax.jit
def kernel(features, edge_index, W1, W2, W3):
    fb = features.astype(jnp.bfloat16).reshape(N_NODES, HALF, 2)
    pt = jax.lax.bitcast_convert_type(
        jax.lax.bitcast_convert_type(fb, jnp.int32), jnp.float32)

    src = edge_index[0]
    dst = edge_index[1]
    pad = E_PAD - N_EDGES
    src_p = jnp.concatenate(
        [src, jnp.zeros((pad,), jnp.int32)]).reshape(-1, CHUNK)
    dst_p = jnp.concatenate(
        [dst, jnp.full((pad,), TRASH_ROW, jnp.int32)]).reshape(-1, CHUNK)

    zrows = jnp.zeros((SLAB, HALF), jnp.float32)

    agg_part, bounce = _sc_a(pt, src_p, dst_p, zrows)
    agg = _sc_b(agg_part, bounce, dst_p)

    return _tc_pool(features, agg[0], agg[1], W1, W2[0::2], W2[1::2], W3)


# speedup vs baseline: 1.5855x; 1.5855x over previous
import functools

import jax
import jax.numpy as jnp
from jax import lax
from jax.experimental import pallas as pl
from jax.experimental.pallas import tpu as pltpu
from jax.experimental.pallas import tpu_sc as plsc

N_NODES = 10000
N_EDGES = 160000
IN_DIM = 256
OUT_DIM = 256

NC = 2
NS = 16
HALF = IN_DIM // 2
CHUNK = 128
CHUNKS_PER_TILE = 80
E_PAD = NS * CHUNKS_PER_TILE * CHUNK
AGG_ROWS = 10008
TRASH_ROW = N_NODES

IDX_GROUP = 32
GROUPS = [(0, 32), (32, 32), (64, 16)]

SLAB = 632
LAST_SLAB = N_NODES - 15 * SLAB
LAST_BASE = 15 * SLAB


def _sc_agg_body(fs, srcs, dsts, zrows, out,
                 srcA, srcB, dstA, dstB, rA, rB, agg_sh,
                 gA, gB, sA, sB, iS):
    rows_b = [rA, rB]
    gsems = [gA, gB]
    ssems = [sA, sB]
    src_pair = [srcA, srcB]
    dst_pair = [dstA, dstB]
    cid = lax.axis_index("c")
    sid = lax.axis_index("s")
    tbase = sid * CHUNKS_PER_TILE

    def load_idx(gi):
        gbase, gsize = GROUPS[gi]
        p = gi % 2
        pltpu.async_copy(srcs.at[pl.ds(tbase + gbase, gsize)],
                         src_pair[p].at[pl.ds(0, gsize)], iS)
        pltpu.async_copy(dsts.at[pl.ds(tbase + gbase, gsize)],
                         dst_pair[p].at[pl.ds(0, gsize)], iS)

    def wait_idx(gi):
        gbase, gsize = GROUPS[gi]
        p = gi % 2
        pltpu.make_async_copy(srcs.at[pl.ds(tbase + gbase, gsize)],
                              src_pair[p].at[pl.ds(0, gsize)], iS).wait()
        pltpu.make_async_copy(dsts.at[pl.ds(tbase + gbase, gsize)],
                              dst_pair[p].at[pl.ds(0, gsize)], iS).wait()

    load_idx(0)

    @pl.when(sid < 15)
    def _():
        pltpu.sync_copy(zrows, agg_sh.at[pl.ds(sid * SLAB, SLAB)])

    @pl.when(sid == 15)
    def _():
        pltpu.sync_copy(zrows.at[pl.ds(0, LAST_SLAB)],
                        agg_sh.at[pl.ds(LAST_BASE, LAST_SLAB)])

    wait_idx(0)
    plsc.subcore_barrier()

    def edge_pass(feat):
        def start_gather(sv, c, b):
            pltpu.async_copy(feat.at[sv.at[c]], rows_b[b], gsems[b])

        def wait_gather(sv, b):
            pltpu.make_async_copy(feat.at[sv.at[0]], rows_b[b],
                                  gsems[b]).wait()

        def start_scatter(dv, c, b):
            pltpu.async_copy(rows_b[b], agg_sh.at[dv.at[c]], ssems[b],
                             add=True)

        def wait_scatter(dv, b):
            pltpu.make_async_copy(rows_b[b], agg_sh.at[dv.at[0]],
                                  ssems[b]).wait()

        for gi, (gbase, gsize) in enumerate(GROUPS):
            sv = src_pair[gi % 2]
            dv = dst_pair[gi % 2]
            if gi + 1 < len(GROUPS):
                load_idx(gi + 1)
            start_gather(sv, 0, 0)
            start_gather(sv, 1, 1)

            def step(jo, carry):
                for b in range(2):
                    c = jo * 2 + b
                    wait_gather(sv, b)
                    start_scatter(dv, c, b)

                    @pl.when(c + 2 < gsize)
                    def _():
                        wait_scatter(dv, b)
                        start_gather(sv, c + 2, b)
                return carry

            lax.fori_loop(0, gsize // 2, step, 0)
            for b in range(2):
                wait_scatter(dv, b)
            if gi + 1 < len(GROUPS):
                wait_idx(gi + 1)

    edge_pass(fs.at[cid])

    plsc.subcore_barrier()

    @pl.when(sid < 15)
    def _():
        pltpu.sync_copy(agg_sh.at[pl.ds(sid * SLAB, SLAB)],
                        out.at[cid, pl.ds(sid * SLAB, SLAB)])

    @pl.when(sid == 15)
    def _():
        pltpu.sync_copy(agg_sh.at[pl.ds(LAST_BASE, LAST_SLAB)],
                        out.at[cid, pl.ds(LAST_BASE, LAST_SLAB)])


_sc_agg = pl.kernel(
    _sc_agg_body,
    out_type=jax.ShapeDtypeStruct((NC, N_NODES, HALF), jnp.float32),
    mesh=plsc.VectorSubcoreMesh(core_axis_name="c", subcore_axis_name="s",
                                num_cores=NC, num_subcores=NS),
    scratch_types=[
        pltpu.VMEM((IDX_GROUP, CHUNK), jnp.int32),
        pltpu.VMEM((IDX_GROUP, CHUNK), jnp.int32),
        pltpu.VMEM((IDX_GROUP, CHUNK), jnp.int32),
        pltpu.VMEM((IDX_GROUP, CHUNK), jnp.int32),
        pltpu.VMEM((CHUNK, HALF), jnp.float32),
        pltpu.VMEM((CHUNK, HALF), jnp.float32),
        pltpu.VMEM_SHARED((AGG_ROWS, HALF), jnp.float32),
    ] + [pltpu.SemaphoreType.DMA] * 5,
)


ROW_BLK = 2000
GRID = N_NODES // ROW_BLK


def _tc_body(f_ref, a0_ref, a1_ref, w1_ref, w2a_ref, w2b_ref, w3_ref,
             out_ref, acc_ref):
    i = pl.program_id(0)
    z = jnp.tanh(
        jnp.dot(f_ref[...], w1_ref[...], preferred_element_type=jnp.float32)
        + jnp.dot(a0_ref[...], w2a_ref[...],
                  preferred_element_type=jnp.float32)
        + jnp.dot(a1_ref[...], w2b_ref[...],
                  preferred_element_type=jnp.float32))
    p = jnp.sum(z, axis=0, keepdims=True)

    @pl.when(i == 0)
    def _():
        acc_ref[...] = p

    @pl.when(i != 0)
    def _():
        acc_ref[...] = acc_ref[...] + p

    @pl.when(i == GRID - 1)
    def _():
        out_ref[...] = jnp.dot(acc_ref[...], w3_ref[...],
                               preferred_element_type=jnp.float32)


_tc_pool = pl.pallas_call(
    _tc_body,
    grid=(GRID,),
    in_specs=[
        pl.BlockSpec((ROW_BLK, IN_DIM), lambda i: (i, 0)),
        pl.BlockSpec((ROW_BLK, HALF), lambda i: (i, 0)),
        pl.BlockSpec((ROW_BLK, HALF), lambda i: (i, 0)),
        pl.BlockSpec((IN_DIM, OUT_DIM), lambda i: (0, 0)),
        pl.BlockSpec((HALF, OUT_DIM), lambda i: (0, 0)),
        pl.BlockSpec((HALF, OUT_DIM), lambda i: (0, 0)),
        pl.BlockSpec((OUT_DIM, OUT_DIM), lambda i: (0, 0)),
    ],
    out_specs=pl.BlockSpec((1, OUT_DIM), lambda i: (0, 0)),
    out_shape=jax.ShapeDtypeStruct((1, OUT_DIM), jnp.float32),
    scratch_shapes=[pltpu.VMEM((1, OUT_DIM), jnp.float32)],
)


@jax.jit
def kernel(features, edge_index, W1, W2, W3):
    f0 = features[:, :HALF]
    f1 = features[:, HALF:]

    src = edge_index[0]
    dst = edge_index[1]
    pad = E_PAD - N_EDGES
    src_p = jnp.concatenate(
        [src, jnp.zeros((pad,), jnp.int32)]).reshape(-1, CHUNK)
    dst_p = jnp.concatenate(
        [dst, jnp.full((pad,), TRASH_ROW, jnp.int32)]).reshape(-1, CHUNK)

    zrows = jnp.zeros((SLAB, HALF), jnp.float32)

    fs = jnp.stack([f0, f1])
    agg = _sc_agg(fs, src_p, dst_p, zrows)

    return _tc_pool(features, agg[0], agg[1], W1, W2[:HALF], W2[HALF:], W3)


# speedup vs baseline: 1.5872x; 1.0011x over previous
import jax
import jax.numpy as jnp
from jax import lax
from jax.experimental import pallas as pl
from jax.experimental.pallas import tpu as pltpu
from jax.experimental.pallas import tpu_sc as plsc

N_NODES = 10000
N_EDGES = 160000
IN_DIM = 256
OUT_DIM = 256

NC = 2
NS = 16
HALF = IN_DIM // 2
CHUNK = 128
CHUNKS_PER_TILE = 80
E_PAD = NS * CHUNKS_PER_TILE * CHUNK
AGG_ROWS = 10008
TRASH_ROW = N_NODES

IDX_GROUP = 32
GROUPS = [(0, 32), (32, 32), (64, 16)]

SLAB = 632
LAST_SLAB = N_NODES - 15 * SLAB
LAST_BASE = 15 * SLAB


def _sc_agg_body(fs, srcs, dsts, zrows, out,
                 srcA, srcB, dstA, dstB, rA, rB, agg_sh,
                 gA, gB, sA, sB, iS):
    rows_b = [rA, rB]
    gsems = [gA, gB]
    ssems = [sA, sB]
    src_pair = [srcA, srcB]
    dst_pair = [dstA, dstB]
    cid = lax.axis_index("c")
    sid = lax.axis_index("s")
    tbase = sid * CHUNKS_PER_TILE

    def load_idx(gi):
        gbase, gsize = GROUPS[gi]
        p = gi % 2
        pltpu.async_copy(srcs.at[pl.ds(tbase + gbase, gsize)],
                         src_pair[p].at[pl.ds(0, gsize)], iS)
        pltpu.async_copy(dsts.at[pl.ds(tbase + gbase, gsize)],
                         dst_pair[p].at[pl.ds(0, gsize)], iS)

    def wait_idx(gi):
        gbase, gsize = GROUPS[gi]
        p = gi % 2
        pltpu.make_async_copy(srcs.at[pl.ds(tbase + gbase, gsize)],
                              src_pair[p].at[pl.ds(0, gsize)], iS).wait()
        pltpu.make_async_copy(dsts.at[pl.ds(tbase + gbase, gsize)],
                              dst_pair[p].at[pl.ds(0, gsize)], iS).wait()

    load_idx(0)

    @pl.when(sid < 15)
    def _():
        pltpu.sync_copy(zrows, agg_sh.at[pl.ds(sid * SLAB, SLAB)])

    @pl.when(sid == 15)
    def _():
        pltpu.sync_copy(zrows.at[pl.ds(0, LAST_SLAB)],
                        agg_sh.at[pl.ds(LAST_BASE, LAST_SLAB)])

    wait_idx(0)
    plsc.subcore_barrier()

    def edge_pass(feat):
        def start_gather(sv, c, b):
            pltpu.async_copy(feat.at[sv.at[c]], rows_b[b], gsems[b])

        def wait_gather(sv, b):
            pltpu.make_async_copy(feat.at[sv.at[0]], rows_b[b],
                                  gsems[b]).wait()

        def start_scatter(dv, c, b):
            pltpu.async_copy(rows_b[b], agg_sh.at[dv.at[c]], ssems[b],
                             add=True)

        def wait_scatter(dv, b):
            pltpu.make_async_copy(rows_b[b], agg_sh.at[dv.at[0]],
                                  ssems[b]).wait()

        for gi, (gbase, gsize) in enumerate(GROUPS):
            sv = src_pair[gi % 2]
            dv = dst_pair[gi % 2]
            if gi + 1 < len(GROUPS):
                load_idx(gi + 1)
            start_gather(sv, 0, 0)
            start_gather(sv, 1, 1)

            def step(jo, carry):
                for b in range(2):
                    c = jo * 2 + b
                    wait_gather(sv, b)
                    start_scatter(dv, c, b)

                    @pl.when(c + 2 < gsize)
                    def _():
                        wait_scatter(dv, b)
                        start_gather(sv, c + 2, b)
                return carry

            lax.fori_loop(0, gsize // 2, step, 0)
            for b in range(2):
                wait_scatter(dv, b)
            if gi + 1 < len(GROUPS):
                wait_idx(gi + 1)

    edge_pass(fs.at[cid])

    plsc.subcore_barrier()

    @pl.when(sid < 15)
    def _():
        pltpu.sync_copy(agg_sh.at[pl.ds(sid * SLAB, SLAB)],
                        out.at[cid, pl.ds(sid * SLAB, SLAB)])

    @pl.when(sid == 15)
    def _():
        pltpu.sync_copy(agg_sh.at[pl.ds(LAST_BASE, LAST_SLAB)],
                        out.at[cid, pl.ds(LAST_BASE, LAST_SLAB)])


_sc_agg = pl.kernel(
    _sc_agg_body,
    out_type=jax.ShapeDtypeStruct((NC, N_NODES, HALF), jnp.float32),
    mesh=plsc.VectorSubcoreMesh(core_axis_name="c", subcore_axis_name="s",
                                num_cores=NC, num_subcores=NS),
    scratch_types=[
        pltpu.VMEM((IDX_GROUP, CHUNK), jnp.int32),
        pltpu.VMEM((IDX_GROUP, CHUNK), jnp.int32),
        pltpu.VMEM((IDX_GROUP, CHUNK), jnp.int32),
        pltpu.VMEM((IDX_GROUP, CHUNK), jnp.int32),
        pltpu.VMEM((CHUNK, HALF), jnp.float32),
        pltpu.VMEM((CHUNK, HALF), jnp.float32),
        pltpu.VMEM_SHARED((AGG_ROWS, HALF), jnp.float32),
    ] + [pltpu.SemaphoreType.DMA] * 5,
)


ROW_BLK = 2000
GRID = N_NODES // ROW_BLK


def _tc_body(f_ref, a0_ref, a1_ref, w1_ref, w2a_ref, w2b_ref, w3_ref,
             out_ref, acc_ref):
    i = pl.program_id(0)
    z = jnp.tanh(
        jnp.dot(f_ref[...], w1_ref[...], preferred_element_type=jnp.float32)
        + jnp.dot(a0_ref[...], w2a_ref[...],
                  preferred_element_type=jnp.float32)
        + jnp.dot(a1_ref[...], w2b_ref[...],
                  preferred_element_type=jnp.float32))
    p = jnp.sum(z, axis=0, keepdims=True)

    @pl.when(i == 0)
    def _():
        acc_ref[...] = p

    @pl.when(i != 0)
    def _():
        acc_ref[...] = acc_ref[...] + p

    @pl.when(i == GRID - 1)
    def _():
        out_ref[...] = jnp.dot(acc_ref[...], w3_ref[...],
                               preferred_element_type=jnp.float32)


_tc_pool = pl.pallas_call(
    _tc_body,
    grid=(GRID,),
    in_specs=[
        pl.BlockSpec((ROW_BLK, IN_DIM), lambda i: (i, 0)),
        pl.BlockSpec((ROW_BLK, HALF), lambda i: (i, 0)),
        pl.BlockSpec((ROW_BLK, HALF), lambda i: (i, 0)),
        pl.BlockSpec((IN_DIM, OUT_DIM), lambda i: (0, 0)),
        pl.BlockSpec((HALF, OUT_DIM), lambda i: (0, 0)),
        pl.BlockSpec((HALF, OUT_DIM), lambda i: (0, 0)),
        pl.BlockSpec((OUT_DIM, OUT_DIM), lambda i: (0, 0)),
    ],
    out_specs=pl.BlockSpec((1, OUT_DIM), lambda i: (0, 0)),
    out_shape=jax.ShapeDtypeStruct((1, OUT_DIM), jnp.float32),
    scratch_shapes=[pltpu.VMEM((1, OUT_DIM), jnp.float32)],
)


@jax.jit
def kernel(features, edge_index, W1, W2, W3):
    f0 = features[:, :HALF]
    f1 = features[:, HALF:]

    src = edge_index[0]
    dst = edge_index[1]
    pad = E_PAD - N_EDGES
    src_p = jnp.concatenate(
        [src, jnp.zeros((pad,), jnp.int32)]).reshape(-1, CHUNK)
    dst_p = jnp.concatenate(
        [dst, jnp.full((pad,), TRASH_ROW, jnp.int32)]).reshape(-1, CHUNK)

    zrows = jnp.zeros((SLAB, HALF), jnp.float32)

    fs = jnp.stack([f0, f1])
    agg = _sc_agg(fs, src_p, dst_p, zrows)

    return _tc_pool(features, agg[0], agg[1], W1, W2[:HALF], W2[HALF:], W3)
